# baseline (speedup 1.0000x reference)
import jax
import jax.numpy as jnp
from jax.experimental import pallas as pl
from jax.experimental.pallas import tpu as pltpu

_MIDI_PITCHES = 88
_MIN_MIDI_PITCH = 21
_C = 11
_HOP_SHIFT = 9
_PC = _MIDI_PITCHES * _C
_NBUF = 4


def _body(nc_ref, tp_ref, out_ref, *bufs_sems):
    n = nc_ref.shape[1]
    t_frames = out_ref.shape[3]
    bufs, sems = bufs_sems[:_NBUF], bufs_sems[_NBUF:]
    i = pl.program_id(0)
    nb = pl.num_programs(0) - 1

    for s in range(_NBUF):
        @pl.when((i >= 1) & (jax.lax.rem(i - 1, _NBUF) == s))
        def _(s=s):
            pltpu.make_async_copy(bufs[s], out_ref.at[pl.ds(i - 1, 1)],
                                  sems[s]).start()

    @pl.when(i < nb)
    def _compute():
        nc = nc_ref[0]
        tp = tp_ref[0]

        pitch_col = nc[:, 0:1] - _MIN_MIDI_PITCH
        start_col = jnp.right_shift(nc[:, 1:2], _HOP_SHIFT)
        end_raw = nc[:, 2:3]
        end_col = jnp.where(end_raw >= 0,
                            jnp.right_shift(end_raw, _HOP_SHIFT), -1)

        tg = jax.lax.broadcasted_iota(jnp.int32, (n, t_frames), 1)
        mask = ((tg >= start_col) & (tg < end_col)).astype(jnp.float32)

        q_row = jax.lax.broadcasted_iota(jnp.int32, (1, _PC), 1)
        pm = (q_row % _MIDI_PITCHES == pitch_col).astype(jnp.float32)
        s_sel = (jax.lax.broadcasted_iota(jnp.int32, (_C, _PC), 0)
                 == jax.lax.broadcasted_iota(jnp.int32, (_C, _PC), 1)
                 // _MIDI_PITCHES).astype(jnp.float32)
        tpsel = jnp.dot(tp, s_sel, preferred_element_type=jnp.float32)
        m_mat = pm * tpsel

        res = jax.lax.dot_general(m_mat, mask, (((0,), (0,)), ((), ())),
                                  preferred_element_type=jnp.float32)
        res3 = res.reshape(1, _C, _MIDI_PITCHES, t_frames)

        for s in range(_NBUF):
            @pl.when(jax.lax.rem(i, _NBUF) == s)
            def _(s=s):
                @pl.when(i >= _NBUF)
                def _():
                    pltpu.make_async_copy(bufs[s],
                                          out_ref.at[pl.ds(i, 1)],
                                          sems[s]).wait()
                bufs[s][...] = res3

    @pl.when(i == nb)
    def _drain():
        for s in range(_NBUF):
            pltpu.make_async_copy(bufs[s], out_ref.at[pl.ds(nb - 1, 1)],
                                  sems[s]).wait()


def kernel(note_croppings, timbre_probs, pianorolls):
    b, n, _ = note_croppings.shape
    t_frames = pianorolls.shape[1]
    last = b - 1
    out = pl.pallas_call(
        _body,
        grid=(b + 1,),
        in_specs=[
            pl.BlockSpec((1, n, 3),
                         lambda i: (jnp.minimum(i, last), 0, 0)),
            pl.BlockSpec((1, n, _C),
                         lambda i: (jnp.minimum(i, last), 0, 0)),
        ],
        out_specs=pl.BlockSpec(memory_space=pltpu.MemorySpace.HBM),
        out_shape=jax.ShapeDtypeStruct((b, _C, _MIDI_PITCHES, t_frames),
                                       jnp.float32),
        scratch_shapes=(
            [pltpu.VMEM((1, _C, _MIDI_PITCHES, t_frames), jnp.float32)]
            * _NBUF
            + [pltpu.SemaphoreType.DMA] * _NBUF),
        compiler_params=pltpu.CompilerParams(
            dimension_semantics=("arbitrary",)),
    )(note_croppings, timbre_probs)
    return out.transpose(0, 3, 2, 1)


# speedup vs baseline: 1.0128x; 1.0128x over previous
import jax
import jax.numpy as jnp
from jax.experimental import pallas as pl
from jax.experimental.pallas import tpu as pltpu

_MIDI_PITCHES = 88
_MIN_MIDI_PITCH = 21
_C = 11
_HOP_SHIFT = 9
_PC = _MIDI_PITCHES * _C


def _body(nc_ref, tp_ref, out_ref):
    n = nc_ref.shape[1]
    t_frames = out_ref.shape[3]
    nc = nc_ref[0]
    tp = tp_ref[0]

    pitch_col = nc[:, 0:1] - _MIN_MIDI_PITCH
    start_col = jnp.right_shift(nc[:, 1:2], _HOP_SHIFT)
    end_raw = nc[:, 2:3]
    end_col = jnp.where(end_raw >= 0,
                        jnp.right_shift(end_raw, _HOP_SHIFT), -1)

    tg = jax.lax.broadcasted_iota(jnp.int32, (n, t_frames), 1)
    mask = ((tg >= start_col) & (tg < end_col)).astype(jnp.float32)

    q_row = jax.lax.broadcasted_iota(jnp.int32, (1, _PC), 1)
    pm = (q_row % _MIDI_PITCHES == pitch_col).astype(jnp.float32)
    s_sel = (jax.lax.broadcasted_iota(jnp.int32, (_C, _PC), 0)
             == jax.lax.broadcasted_iota(jnp.int32, (_C, _PC), 1)
             // _MIDI_PITCHES).astype(jnp.float32)
    tpsel = jnp.dot(tp, s_sel, preferred_element_type=jnp.float32)
    m_mat = pm * tpsel

    res = jax.lax.dot_general(m_mat, mask, (((0,), (0,)), ((), ())),
                              preferred_element_type=jnp.float32)
    out_ref[0] = res.reshape(_C, _MIDI_PITCHES, t_frames)


def kernel(note_croppings, timbre_probs, pianorolls):
    b, n, _ = note_croppings.shape
    t_frames = pianorolls.shape[1]
    out = pl.pallas_call(
        _body,
        grid=(b,),
        in_specs=[
            pl.BlockSpec((1, n, 3), lambda i: (i, 0, 0)),
            pl.BlockSpec((1, n, _C), lambda i: (i, 0, 0)),
        ],
        out_specs=pl.BlockSpec((1, _C, _MIDI_PITCHES, t_frames),
                               lambda i: (i, 0, 0, 0)),
        out_shape=jax.ShapeDtypeStruct((b, _C, _MIDI_PITCHES, t_frames),
                                       jnp.float32),
        compiler_params=pltpu.CompilerParams(
            dimension_semantics=("parallel",)),
    )(note_croppings, timbre_probs)
    return out.transpose(0, 3, 2, 1)
